# baseline (speedup 1.0000x reference)
import functools

import jax
import jax.numpy as jnp
from jax import lax
from jax.experimental import pallas as pl
from jax.experimental.pallas import tpu as pltpu
from jax.experimental.pallas import tpu_sc as plsc

NC = 2
NS = 16
NW = NC * NS
LANES = 16

CHUNK = 48


def _sc_mesh():
    return plsc.VectorSubcoreMesh(
        core_axis_name="c", subcore_axis_name="s", num_cores=NC, num_subcores=NS
    )


def _sc_hist(dst, n_nodes, n_bins):
    (E,) = dst.shape
    e_per_w = E // NW
    assert E % NW == 0 and e_per_w % LANES == 0 and n_bins % LANES == 0

    @functools.partial(
        pl.kernel,
        out_type=jax.ShapeDtypeStruct((NW, n_bins), jnp.float32),
        mesh=_sc_mesh(),
        compiler_params=pltpu.CompilerParams(needs_layout_passes=False),
        scratch_types=[
            pltpu.VMEM((e_per_w,), jnp.int32),
            pltpu.VMEM((n_bins,), jnp.float32),
        ],
    )
    def hist_kernel(dst_hbm, out_hbm, dst_v, hist_v):
        wid = lax.axis_index("s") * NC + lax.axis_index("c")
        pltpu.sync_copy(dst_hbm.at[pl.ds(wid * e_per_w, e_per_w)], dst_v)
        zeros16 = jnp.zeros((LANES,), jnp.float32)

        def zero_body(i, _):
            hist_v[pl.ds(i * LANES, LANES)] = zeros16
            return 0

        lax.fori_loop(0, n_bins // LANES, zero_body, 0)
        ones16 = jnp.ones((LANES,), jnp.float32)

        def acc_body(i, _):
            idx = dst_v[pl.ds(i * LANES, LANES)]
            plsc.addupdate_scatter(hist_v, [idx], ones16)
            return 0

        lax.fori_loop(0, e_per_w // LANES, acc_body, 0)
        pltpu.sync_copy(hist_v, out_hbm.at[wid])

    return hist_kernel(dst)


NBUF = 4


def _sc_aggregate(y, src3, dst3, zeros_init):
    n_nodes, feat = y.shape
    cpw = src3.shape[1] // CHUNK
    n_acc = zeros_init.shape[0]
    rpt = n_acc // NS
    assert cpw % NBUF == 0 and n_acc % NS == 0 and rpt % 8 == 0

    @functools.partial(
        pl.kernel,
        out_type=jax.ShapeDtypeStruct((NC, n_acc, feat), jnp.float32),
        mesh=_sc_mesh(),
        compiler_params=pltpu.CompilerParams(needs_layout_passes=False),
        scratch_types=[
            pltpu.VMEM((cpw * CHUNK,), jnp.int32),
            pltpu.VMEM((cpw * CHUNK,), jnp.int32),
            pltpu.VMEM((NBUF, CHUNK, feat), jnp.float32),
            pltpu.VMEM_SHARED((n_acc, feat), jnp.float32),
            pltpu.SemaphoreType.DMA((NBUF,)),
            pltpu.SemaphoreType.DMA((NBUF,)),
        ],
    )
    def agg_kernel(y_hbm, src_hbm, dst_hbm, zero_hbm, out_hbm, src_v, dst_v,
                   rows_v, acc_sh, gsem, ssem):
        cid = lax.axis_index("c")
        sid = lax.axis_index("s")
        wid = sid * NC + cid

        pltpu.sync_copy(src_hbm.at[wid], src_v)
        pltpu.sync_copy(dst_hbm.at[wid], dst_v)
        pltpu.sync_copy(
            zero_hbm.at[pl.ds(sid * rpt, rpt)], acc_sh.at[pl.ds(sid * rpt, rpt)]
        )
        plsc.subcore_barrier()

        def start_gather(b, j):
            pltpu.async_copy(
                y_hbm.at[src_v.at[pl.ds(j * CHUNK, CHUNK)]], rows_v.at[b], gsem.at[b]
            )

        for b in range(NBUF):
            start_gather(b, b)

        def group_body(g, _):
            sdescs = []
            for b in range(NBUF):
                j = g * NBUF + b
                pltpu.make_async_copy(
                    y_hbm.at[pl.ds(0, CHUNK)], rows_v.at[b], gsem.at[b]
                ).wait()
                sdescs.append(
                    pltpu.async_copy(
                        rows_v.at[b],
                        acc_sh.at[dst_v.at[pl.ds(j * CHUNK, CHUNK)]],
                        ssem.at[b],
                        add=True,
                    )
                )
            for b in range(NBUF):
                sdescs[b].wait()

                @pl.when(g < cpw // NBUF - 1)
                def _():
                    start_gather(b, g * NBUF + b + NBUF)

            return 0

        lax.fori_loop(0, cpw // NBUF, group_body, 0)
        plsc.subcore_barrier()

        pltpu.sync_copy(
            acc_sh.at[pl.ds(sid * rpt, rpt)], out_hbm.at[cid, pl.ds(sid * rpt, rpt)]
        )

    return agg_kernel(y, src3, dst3, zeros_init)


def _tc_prescale(hist_t, x, w1):
    n_nodes, gene = x.shape
    hidden = w1.shape[1]
    blk = 2000
    nblk = n_nodes // blk
    assert n_nodes % blk == 0

    def body(hist_ref, x_ref, w1_ref, y1_ref, dinv_ref):
        deg = jnp.sum(hist_ref[...], axis=1, keepdims=True) + 1.0
        dinv = lax.rsqrt(deg)
        dinv_ref[...] = dinv
        y1_ref[...] = dinv * jnp.dot(
            x_ref[...], w1_ref[...], preferred_element_type=jnp.float32
        )

    return pl.pallas_call(
        body,
        grid=(nblk,),
        in_specs=[
            pl.BlockSpec((blk, NW), lambda i: (i, 0)),
            pl.BlockSpec((blk, gene), lambda i: (i, 0)),
            pl.BlockSpec((gene, hidden), lambda i: (0, 0)),
        ],
        out_specs=[
            pl.BlockSpec((blk, hidden), lambda i: (i, 0)),
            pl.BlockSpec((blk, 1), lambda i: (i, 0)),
        ],
        out_shape=[
            jax.ShapeDtypeStruct((n_nodes, hidden), jnp.float32),
            jax.ShapeDtypeStruct((n_nodes, 1), jnp.float32),
        ],
    )(hist_t, x, w1)


def _tc_mid(z1p, y1, dinv, b1, w2):
    n_nodes, hidden = y1.shape
    feat = w2.shape[1]
    blk = 2000
    nblk = n_nodes // blk

    def body(z_ref, y1_ref, dinv_ref, b1_ref, w2_ref, y2_ref):
        dinv = dinv_ref[...]
        s1 = jax.nn.relu(dinv * (z_ref[0] + z_ref[1] + y1_ref[...]) + b1_ref[...])
        y2_ref[...] = dinv * jnp.dot(
            s1, w2_ref[...], preferred_element_type=jnp.float32
        )

    return pl.pallas_call(
        body,
        grid=(nblk,),
        in_specs=[
            pl.BlockSpec((NC, blk, hidden), lambda i: (0, i, 0)),
            pl.BlockSpec((blk, hidden), lambda i: (i, 0)),
            pl.BlockSpec((blk, 1), lambda i: (i, 0)),
            pl.BlockSpec((1, hidden), lambda i: (0, 0)),
            pl.BlockSpec((hidden, feat), lambda i: (0, 0)),
        ],
        out_specs=pl.BlockSpec((blk, feat), lambda i: (i, 0)),
        out_shape=jax.ShapeDtypeStruct((n_nodes, feat), jnp.float32),
    )(z1p, y1, dinv, b1, w2)


def _tc_final(z2p, y2, dinv, b2, data_t):
    n_nodes, feat = y2.shape
    batch = data_t.shape[1]
    blk = 2000
    nblk = n_nodes // blk

    def body(z_ref, y2_ref, dinv_ref, b2_ref, data_ref, out_ref):
        h2 = dinv_ref[...] * (z_ref[0] + z_ref[1] + y2_ref[...]) + b2_ref[...]
        part = lax.dot_general(
            data_ref[...], h2, (((0,), (0,)), ((), ())),
            preferred_element_type=jnp.float32,
        )

        @pl.when(pl.program_id(0) == 0)
        def _():
            out_ref[...] = jnp.zeros_like(out_ref)

        out_ref[...] += part

    return pl.pallas_call(
        body,
        grid=(nblk,),
        in_specs=[
            pl.BlockSpec((NC, blk, feat), lambda i: (0, i, 0)),
            pl.BlockSpec((blk, feat), lambda i: (i, 0)),
            pl.BlockSpec((blk, 1), lambda i: (i, 0)),
            pl.BlockSpec((1, feat), lambda i: (0, 0)),
            pl.BlockSpec((blk, batch), lambda i: (i, 0)),
        ],
        out_specs=pl.BlockSpec((batch, feat), lambda i: (0, 0)),
        out_shape=jax.ShapeDtypeStruct((batch, feat), jnp.float32),
    )(z2p, y2, dinv, b2, data_t)


def kernel(data, x, edge_index, W1, b1, W2, b2):
    n_nodes = x.shape[0]
    E = edge_index.shape[1]

    cpw = -(-E // (NW * CHUNK) // NBUF) * NBUF
    e_pad = NW * cpw * CHUNK
    n_acc = -(-n_nodes // (8 * NS)) * (8 * NS)
    pad = e_pad - E
    src = jnp.concatenate([edge_index[0], jnp.zeros((pad,), jnp.int32)])
    pad_dst = n_nodes + (jnp.arange(pad, dtype=jnp.int32) % (n_acc - n_nodes))
    dst = jnp.concatenate([edge_index[1], pad_dst])
    src3 = src.reshape(NW, cpw * CHUNK)
    dst3 = dst.reshape(NW, cpw * CHUNK)

    zeros_init = jnp.zeros((n_acc, x.shape[1]), jnp.float32)
    hist = _sc_hist(dst, n_nodes, n_acc)
    hist_t = hist[:, :n_nodes].T

    y1, dinv = _tc_prescale(hist_t, x, W1)
    z1p = _sc_aggregate(y1, src3, dst3, zeros_init)
    y2 = _tc_mid(z1p, y1, dinv, b1.reshape(1, -1), W2)
    z2p = _sc_aggregate(y2, src3, dst3, zeros_init)
    return _tc_final(z2p, y2, dinv, b2.reshape(1, -1), data.T)


# speedup vs baseline: 1.2053x; 1.2053x over previous
import functools

import jax
import jax.numpy as jnp
from jax import lax
from jax.experimental import pallas as pl
from jax.experimental.pallas import tpu as pltpu
from jax.experimental.pallas import tpu_sc as plsc

NC = 2
NS = 16
NW = NC * NS
LANES = 16

CHUNK = 80


def _sc_mesh():
    return plsc.VectorSubcoreMesh(
        core_axis_name="c", subcore_axis_name="s", num_cores=NC, num_subcores=NS
    )


def _sc_hist(dst, n_nodes):
    (E,) = dst.shape
    e_per_w = E // NW
    assert E % NW == 0 and e_per_w % LANES == 0

    @functools.partial(
        pl.kernel,
        out_type=jax.ShapeDtypeStruct((NW, n_nodes), jnp.float32),
        mesh=_sc_mesh(),
        compiler_params=pltpu.CompilerParams(needs_layout_passes=False),
        scratch_types=[
            pltpu.VMEM((e_per_w,), jnp.int32),
            pltpu.VMEM((n_nodes,), jnp.float32),
        ],
    )
    def hist_kernel(dst_hbm, out_hbm, dst_v, hist_v):
        wid = lax.axis_index("s") * NC + lax.axis_index("c")
        pltpu.sync_copy(dst_hbm.at[pl.ds(wid * e_per_w, e_per_w)], dst_v)
        zeros16 = jnp.zeros((LANES,), jnp.float32)

        def zero_body(i, _):
            hist_v[pl.ds(i * LANES, LANES)] = zeros16
            return 0

        lax.fori_loop(0, n_nodes // LANES, zero_body, 0)
        ones16 = jnp.ones((LANES,), jnp.float32)

        def acc_body(i, _):
            idx = dst_v[pl.ds(i * LANES, LANES)]
            plsc.addupdate_scatter(hist_v, [idx], ones16)
            return 0

        lax.fori_loop(0, e_per_w // LANES, acc_body, 0)
        pltpu.sync_copy(hist_v, out_hbm.at[wid])

    return hist_kernel(dst)


def _sc_aggregate(y, src, dst):
    n_nodes, feat = y.shape
    (E,) = src.shape
    e_per_w = E // NW
    n_chunks = e_per_w // CHUNK
    zrows = 80
    n_zchunks = n_nodes // zrows
    zrounds = (n_zchunks + NS - 1) // NS
    assert E % NW == 0 and e_per_w % CHUNK == 0 and n_nodes % zrows == 0

    @functools.partial(
        pl.kernel,
        out_type=jax.ShapeDtypeStruct((NC, n_nodes, feat), jnp.float32),
        mesh=_sc_mesh(),
        compiler_params=pltpu.CompilerParams(needs_layout_passes=False),
        scratch_types=[
            pltpu.VMEM((CHUNK,), jnp.int32),
            pltpu.VMEM((CHUNK,), jnp.int32),
            pltpu.VMEM((CHUNK, feat), jnp.float32),
            pltpu.VMEM((zrows, feat), jnp.float32),
            pltpu.VMEM_SHARED((n_nodes, feat), jnp.float32),
            pltpu.SemaphoreType.DMA,
        ],
    )
    def agg_kernel(y_hbm, src_hbm, dst_hbm, out_hbm, src_v, dst_v, rows_v, zbuf_v, acc_sh, sem):
        cid = lax.axis_index("c")
        sid = lax.axis_index("s")
        wid = sid * NC + cid

        zeros16 = jnp.zeros((LANES,), jnp.float32)

        def zbuf_body(r, _):
            for cc in range(feat // LANES):
                zbuf_v[r, pl.ds(cc * LANES, LANES)] = zeros16
            return 0

        lax.fori_loop(0, zrows, zbuf_body, 0)

        def zcopy_body(k, _):
            chunk = sid + k * NS

            @pl.when(chunk < n_zchunks)
            def _():
                pltpu.sync_copy(zbuf_v, acc_sh.at[pl.ds(chunk * zrows, zrows)])

            return 0

        lax.fori_loop(0, zrounds, zcopy_body, 0)
        plsc.subcore_barrier()

        def edge_body(j, _):
            base = wid * e_per_w + j * CHUNK
            pltpu.sync_copy(src_hbm.at[pl.ds(base, CHUNK)], src_v)
            pltpu.sync_copy(dst_hbm.at[pl.ds(base, CHUNK)], dst_v)
            pltpu.async_copy(y_hbm.at[src_v], rows_v, sem).wait()
            pltpu.sync_copy(rows_v, acc_sh.at[dst_v], add=True)
            return 0

        lax.fori_loop(0, n_chunks, edge_body, 0)
        plsc.subcore_barrier()

        def out_body(k, _):
            chunk = sid + k * NS

            @pl.when(chunk < n_zchunks)
            def _():
                base = chunk * zrows
                pltpu.sync_copy(
                    acc_sh.at[pl.ds(base, zrows)], out_hbm.at[cid, pl.ds(base, zrows)]
                )

            return 0

        lax.fori_loop(0, zrounds, out_body, 0)

    return agg_kernel(y, src, dst)


def _tc_prescale(hist_t, x, w1):
    n_nodes, gene = x.shape
    hidden = w1.shape[1]
    blk = 2000
    nblk = n_nodes // blk
    assert n_nodes % blk == 0

    def body(hist_ref, x_ref, w1_ref, y1_ref, dinv_ref):
        deg = jnp.sum(hist_ref[...], axis=1, keepdims=True) + 1.0
        dinv = lax.rsqrt(deg)
        dinv_ref[...] = dinv
        y1_ref[...] = dinv * jnp.dot(
            x_ref[...], w1_ref[...], preferred_element_type=jnp.float32
        )

    return pl.pallas_call(
        body,
        grid=(nblk,),
        in_specs=[
            pl.BlockSpec((blk, NW), lambda i: (i, 0)),
            pl.BlockSpec((blk, gene), lambda i: (i, 0)),
            pl.BlockSpec((gene, hidden), lambda i: (0, 0)),
        ],
        out_specs=[
            pl.BlockSpec((blk, hidden), lambda i: (i, 0)),
            pl.BlockSpec((blk, 1), lambda i: (i, 0)),
        ],
        out_shape=[
            jax.ShapeDtypeStruct((n_nodes, hidden), jnp.float32),
            jax.ShapeDtypeStruct((n_nodes, 1), jnp.float32),
        ],
    )(hist_t, x, w1)


def _tc_mid(z1p, y1, dinv, b1, w2):
    n_nodes, hidden = y1.shape
    feat = w2.shape[1]
    blk = 2000
    nblk = n_nodes // blk

    def body(z_ref, y1_ref, dinv_ref, b1_ref, w2_ref, y2_ref):
        dinv = dinv_ref[...]
        s1 = jax.nn.relu(dinv * (z_ref[0] + z_ref[1] + y1_ref[...]) + b1_ref[...])
        y2_ref[...] = dinv * jnp.dot(
            s1, w2_ref[...], preferred_element_type=jnp.float32
        )

    return pl.pallas_call(
        body,
        grid=(nblk,),
        in_specs=[
            pl.BlockSpec((NC, blk, hidden), lambda i: (0, i, 0)),
            pl.BlockSpec((blk, hidden), lambda i: (i, 0)),
            pl.BlockSpec((blk, 1), lambda i: (i, 0)),
            pl.BlockSpec((1, hidden), lambda i: (0, 0)),
            pl.BlockSpec((hidden, feat), lambda i: (0, 0)),
        ],
        out_specs=pl.BlockSpec((blk, feat), lambda i: (i, 0)),
        out_shape=jax.ShapeDtypeStruct((n_nodes, feat), jnp.float32),
    )(z1p, y1, dinv, b1, w2)


def _tc_final(z2p, y2, dinv, b2, data_t):
    n_nodes, feat = y2.shape
    batch = data_t.shape[1]
    blk = 2000
    nblk = n_nodes // blk

    def body(z_ref, y2_ref, dinv_ref, b2_ref, data_ref, out_ref):
        h2 = dinv_ref[...] * (z_ref[0] + z_ref[1] + y2_ref[...]) + b2_ref[...]
        part = lax.dot_general(
            data_ref[...], h2, (((0,), (0,)), ((), ())),
            preferred_element_type=jnp.float32,
        )

        @pl.when(pl.program_id(0) == 0)
        def _():
            out_ref[...] = jnp.zeros_like(out_ref)

        out_ref[...] += part

    return pl.pallas_call(
        body,
        grid=(nblk,),
        in_specs=[
            pl.BlockSpec((NC, blk, feat), lambda i: (0, i, 0)),
            pl.BlockSpec((blk, feat), lambda i: (i, 0)),
            pl.BlockSpec((blk, 1), lambda i: (i, 0)),
            pl.BlockSpec((1, feat), lambda i: (0, 0)),
            pl.BlockSpec((blk, batch), lambda i: (i, 0)),
        ],
        out_specs=pl.BlockSpec((batch, feat), lambda i: (0, 0)),
        out_shape=jax.ShapeDtypeStruct((batch, feat), jnp.float32),
    )(z2p, y2, dinv, b2, data_t)


def kernel(data, x, edge_index, W1, b1, W2, b2):
    n_nodes = x.shape[0]
    src = edge_index[0]
    dst = edge_index[1]

    hist = _sc_hist(dst, n_nodes)
    hist_t = hist.T

    y1, dinv = _tc_prescale(hist_t, x, W1)
    z1p = _sc_aggregate(y1, src, dst)
    y2 = _tc_mid(z1p, y1, dinv, b1.reshape(1, -1), W2)
    z2p = _sc_aggregate(y2, src, dst)
    return _tc_final(z2p, y2, dinv, b2.reshape(1, -1), data.T)


# speedup vs baseline: 1.3057x; 1.0833x over previous
import functools

import jax
import jax.numpy as jnp
from jax import lax
from jax.experimental import pallas as pl
from jax.experimental.pallas import tpu as pltpu
from jax.experimental.pallas import tpu_sc as plsc

NC = 2
NS = 16
NW = NC * NS
LANES = 16

CHUNK = 48


def _sc_mesh():
    return plsc.VectorSubcoreMesh(
        core_axis_name="c", subcore_axis_name="s", num_cores=NC, num_subcores=NS
    )


def _sc_hist(dst, n_nodes, n_bins):
    (E,) = dst.shape
    e_per_w = E // NW
    assert E % NW == 0 and e_per_w % LANES == 0 and n_bins % LANES == 0

    @functools.partial(
        pl.kernel,
        out_type=jax.ShapeDtypeStruct((NW, n_bins), jnp.float32),
        mesh=_sc_mesh(),
        compiler_params=pltpu.CompilerParams(needs_layout_passes=False),
        scratch_types=[
            pltpu.VMEM((e_per_w,), jnp.int32),
            pltpu.VMEM((n_bins,), jnp.float32),
        ],
    )
    def hist_kernel(dst_hbm, out_hbm, dst_v, hist_v):
        wid = lax.axis_index("s") * NC + lax.axis_index("c")
        pltpu.sync_copy(dst_hbm.at[pl.ds(wid * e_per_w, e_per_w)], dst_v)
        zeros16 = jnp.zeros((LANES,), jnp.float32)

        def zero_body(i, _):
            hist_v[pl.ds(i * LANES, LANES)] = zeros16
            return 0

        lax.fori_loop(0, n_bins // LANES, zero_body, 0)
        ones16 = jnp.ones((LANES,), jnp.float32)

        def acc_body(i, _):
            idx = dst_v[pl.ds(i * LANES, LANES)]
            plsc.addupdate_scatter(hist_v, [idx], ones16)
            return 0

        lax.fori_loop(0, e_per_w // LANES, acc_body, 0)
        pltpu.sync_copy(hist_v, out_hbm.at[wid])

    return hist_kernel(dst)


NBUF = 3


def _sc_aggregate(y, src3, dst3, zeros_init, cpw_f, cpw_s):
    n_nodes, feat = y.shape
    cpw = src3.shape[1] // CHUNK
    n_acc = zeros_init.shape[0]
    rpt = n_acc // NS
    assert cpw_f % NBUF == 0 and cpw_s % NBUF == 0 and cpw == cpw_f
    assert n_acc % NS == 0 and rpt % 8 == 0

    @functools.partial(
        pl.kernel,
        out_type=jax.ShapeDtypeStruct((NC, n_acc, feat), jnp.float32),
        mesh=_sc_mesh(),
        compiler_params=pltpu.CompilerParams(needs_layout_passes=False),
        scratch_types=[
            pltpu.VMEM((cpw * CHUNK,), jnp.int32),
            pltpu.VMEM((cpw * CHUNK,), jnp.int32),
            pltpu.VMEM((NBUF, CHUNK, feat), jnp.float32),
            pltpu.VMEM_SHARED((n_acc, feat), jnp.float32),
            pltpu.SemaphoreType.DMA((NBUF,)),
            pltpu.SemaphoreType.DMA((NBUF,)),
        ],
    )
    def agg_kernel(y_hbm, src_hbm, dst_hbm, zero_hbm, out_hbm, src_v, dst_v,
                   rows_v, acc_sh, gsem, ssem):
        cid = lax.axis_index("c")
        sid = lax.axis_index("s")
        wid = sid * NC + cid
        ngroups = jnp.where(cid == 0, cpw_f // NBUF, cpw_s // NBUF)

        pltpu.sync_copy(src_hbm.at[wid], src_v)
        pltpu.sync_copy(dst_hbm.at[wid], dst_v)
        pltpu.sync_copy(
            zero_hbm.at[pl.ds(sid * rpt, rpt)], acc_sh.at[pl.ds(sid * rpt, rpt)]
        )
        plsc.subcore_barrier()

        def start_gather(b, j):
            pltpu.async_copy(
                y_hbm.at[src_v.at[pl.ds(j * CHUNK, CHUNK)]], rows_v.at[b], gsem.at[b]
            )

        for b in range(NBUF):
            start_gather(b, b)

        def group_body(g, _):
            sdescs = []
            for b in range(NBUF):
                j = g * NBUF + b
                pltpu.make_async_copy(
                    y_hbm.at[pl.ds(0, CHUNK)], rows_v.at[b], gsem.at[b]
                ).wait()
                sdescs.append(
                    pltpu.async_copy(
                        rows_v.at[b],
                        acc_sh.at[dst_v.at[pl.ds(j * CHUNK, CHUNK)]],
                        ssem.at[b],
                        add=True,
                    )
                )
            for b in range(NBUF):
                sdescs[b].wait()

                @pl.when(g < ngroups - 1)
                def _():
                    start_gather(b, g * NBUF + b + NBUF)

            return 0

        lax.fori_loop(0, ngroups, group_body, 0)
        plsc.subcore_barrier()

        pltpu.sync_copy(
            acc_sh.at[pl.ds(sid * rpt, rpt)], out_hbm.at[cid, pl.ds(sid * rpt, rpt)]
        )

    return agg_kernel(y, src3, dst3, zeros_init)


def _tc_prescale(hist_t, x, w1):
    n_nodes, gene = x.shape
    hidden = w1.shape[1]
    blk = 2000
    nblk = n_nodes // blk
    assert n_nodes % blk == 0

    def body(hist_ref, x_ref, w1_ref, y1_ref, dinv_ref):
        deg = jnp.sum(hist_ref[...], axis=1, keepdims=True) + 1.0
        dinv = lax.rsqrt(deg)
        dinv_ref[...] = dinv
        y1_ref[...] = dinv * jnp.dot(
            x_ref[...], w1_ref[...], preferred_element_type=jnp.float32
        )

    return pl.pallas_call(
        body,
        grid=(nblk,),
        in_specs=[
            pl.BlockSpec((blk, NW), lambda i: (i, 0)),
            pl.BlockSpec((blk, gene), lambda i: (i, 0)),
            pl.BlockSpec((gene, hidden), lambda i: (0, 0)),
        ],
        out_specs=[
            pl.BlockSpec((blk, hidden), lambda i: (i, 0)),
            pl.BlockSpec((blk, 1), lambda i: (i, 0)),
        ],
        out_shape=[
            jax.ShapeDtypeStruct((n_nodes, hidden), jnp.float32),
            jax.ShapeDtypeStruct((n_nodes, 1), jnp.float32),
        ],
    )(hist_t, x, w1)


def _tc_mid(z1p, y1, dinv, b1, w2):
    n_nodes, hidden = y1.shape
    feat = w2.shape[1]
    blk = 2000
    nblk = n_nodes // blk

    def body(z_ref, y1_ref, dinv_ref, b1_ref, w2_ref, y2_ref):
        dinv = dinv_ref[...]
        s1 = jax.nn.relu(dinv * (z_ref[0] + z_ref[1] + y1_ref[...]) + b1_ref[...])
        y2_ref[...] = dinv * jnp.dot(
            s1, w2_ref[...], preferred_element_type=jnp.float32
        )

    return pl.pallas_call(
        body,
        grid=(nblk,),
        in_specs=[
            pl.BlockSpec((NC, blk, hidden), lambda i: (0, i, 0)),
            pl.BlockSpec((blk, hidden), lambda i: (i, 0)),
            pl.BlockSpec((blk, 1), lambda i: (i, 0)),
            pl.BlockSpec((1, hidden), lambda i: (0, 0)),
            pl.BlockSpec((hidden, feat), lambda i: (0, 0)),
        ],
        out_specs=pl.BlockSpec((blk, feat), lambda i: (i, 0)),
        out_shape=jax.ShapeDtypeStruct((n_nodes, feat), jnp.float32),
    )(z1p, y1, dinv, b1, w2)


def _tc_final(z2p, y2, dinv, b2, data_t):
    n_nodes, feat = y2.shape
    batch = data_t.shape[1]
    blk = 2000
    nblk = n_nodes // blk

    def body(z_ref, y2_ref, dinv_ref, b2_ref, data_ref, out_ref):
        h2 = dinv_ref[...] * (z_ref[0] + z_ref[1] + y2_ref[...]) + b2_ref[...]
        part = lax.dot_general(
            data_ref[...], h2, (((0,), (0,)), ((), ())),
            preferred_element_type=jnp.float32,
        )

        @pl.when(pl.program_id(0) == 0)
        def _():
            out_ref[...] = jnp.zeros_like(out_ref)

        out_ref[...] += part

    return pl.pallas_call(
        body,
        grid=(nblk,),
        in_specs=[
            pl.BlockSpec((NC, blk, feat), lambda i: (0, i, 0)),
            pl.BlockSpec((blk, feat), lambda i: (i, 0)),
            pl.BlockSpec((blk, 1), lambda i: (i, 0)),
            pl.BlockSpec((1, feat), lambda i: (0, 0)),
            pl.BlockSpec((blk, batch), lambda i: (i, 0)),
        ],
        out_specs=pl.BlockSpec((batch, feat), lambda i: (0, 0)),
        out_shape=jax.ShapeDtypeStruct((batch, feat), jnp.float32),
    )(z2p, y2, dinv, b2, data_t)


def kernel(data, x, edge_index, W1, b1, W2, b2):
    n_nodes = x.shape[0]
    E = edge_index.shape[1]
    n_acc = -(-n_nodes // (8 * NS)) * (8 * NS)

    e_fast = (E * 3) // 4
    cpw_f = -(-e_fast // (NS * CHUNK) // NBUF) * NBUF
    cpw_s = -(-(E - e_fast) // (NS * CHUNK) // NBUF) * NBUF
    L = cpw_f * CHUNK

    def pack(s_part, d_part, cpw_used):
        used = NS * cpw_used * CHUNK
        pad = used - s_part.shape[0]
        s2 = jnp.concatenate([s_part, jnp.zeros((pad,), jnp.int32)])
        pad_dst = n_nodes + (jnp.arange(pad, dtype=jnp.int32) % (n_acc - n_nodes))
        d2 = jnp.concatenate([d_part, pad_dst])
        s2 = s2.reshape(NS, cpw_used * CHUNK)
        d2 = d2.reshape(NS, cpw_used * CHUNK)
        tail = L - cpw_used * CHUNK
        if tail:
            zs = jnp.zeros((NS, tail), jnp.int32)
            s2 = jnp.concatenate([s2, zs], axis=1)
            d2 = jnp.concatenate([d2, zs], axis=1)
        return s2, d2

    src = edge_index[0]
    dst = edge_index[1]
    src_f, dst_f = pack(src[:e_fast], dst[:e_fast], cpw_f)
    src_s, dst_s = pack(src[e_fast:], dst[e_fast:], cpw_s)
    src3 = jnp.stack([src_f, src_s], axis=1).reshape(NW, L)
    dst3 = jnp.stack([dst_f, dst_s], axis=1).reshape(NW, L)

    zeros_init = jnp.zeros((n_acc, x.shape[1]), jnp.float32)
    hist = _sc_hist(dst, n_nodes, n_nodes)
    hist_t = hist.T

    y1, dinv = _tc_prescale(hist_t, x, W1)
    z1p = _sc_aggregate(y1, src3, dst3, zeros_init, cpw_f, cpw_s)
    y2 = _tc_mid(z1p, y1, dinv, b1.reshape(1, -1), W2)
    z2p = _sc_aggregate(y2, src3, dst3, zeros_init, cpw_f, cpw_s)
    return _tc_final(z2p, y2, dinv, b2.reshape(1, -1), data.T)
